# baseline (speedup 1.0000x reference)
import jax
import jax.numpy as jnp
from jax import lax
from jax.experimental import pallas as pl
from jax.experimental.pallas import tpu as pltpu
from jax.experimental.pallas import tpu_sc as plsc

VOCAB_ = 10000
EMB_ = 32
BATCH_ = 16384

_NC = 2
_NS = 16
_NW = _NC * _NS
_BPW = BATCH_ // _NW
_CH = 128
_NCHUNK = _BPW // _CH


def _gather_body(uid_ref, mid_ref, utab_ref, mtab_ref, out_ref,
                 uidx, midx, urows, mrows, usem, msem):
    wid = lax.axis_index("s") * _NC + lax.axis_index("c")
    base = wid * _BPW
    pltpu.sync_copy(uid_ref.at[pl.ds(base, _BPW)], uidx)
    pltpu.sync_copy(mid_ref.at[pl.ds(base, _BPW)], midx)
    ucopies = [
        pltpu.async_copy(utab_ref.at[uidx.at[pl.ds(j * _CH, _CH)]],
                         urows.at[pl.ds(j * _CH, _CH)], usem)
        for j in range(_NCHUNK)
    ]
    mcopies = [
        pltpu.async_copy(mtab_ref.at[midx.at[pl.ds(j * _CH, _CH)]],
                         mrows.at[pl.ds(j * _CH, _CH)], msem)
        for j in range(_NCHUNK)
    ]
    for c in ucopies:
        c.wait()
    pltpu.sync_copy(urows, out_ref.at[pl.ds(base, _BPW), pl.ds(0, EMB_)])
    for c in mcopies:
        c.wait()
    pltpu.sync_copy(mrows, out_ref.at[pl.ds(base, _BPW), pl.ds(EMB_, EMB_)])


_gather = pl.kernel(
    _gather_body,
    mesh=plsc.VectorSubcoreMesh(core_axis_name="c", subcore_axis_name="s"),
    out_type=jax.ShapeDtypeStruct((BATCH_, 2 * EMB_), jnp.float32),
    scratch_types=[
        pltpu.VMEM((_BPW,), jnp.int32),
        pltpu.VMEM((_BPW,), jnp.int32),
        pltpu.VMEM((_BPW, EMB_), jnp.float32),
        pltpu.VMEM((_BPW, EMB_), jnp.float32),
        pltpu.SemaphoreType.DMA,
        pltpu.SemaphoreType.DMA,
    ],
    compiler_params=pltpu.CompilerParams(use_tc_tiling_on_sc=False),
)

_BT = 2048


def _mlp_body(x_ref, w0_ref, b0_ref, w1_ref, b1_ref, w2t_ref, b2_ref,
              out_ref):
    h = jnp.dot(x_ref[...], w0_ref[...], preferred_element_type=jnp.float32)
    h = jnp.maximum(h + b0_ref[...][None, :], 0.0)
    h = jnp.dot(h, w1_ref[...], preferred_element_type=jnp.float32)
    h = jnp.maximum(h + b1_ref[...][None, :], 0.0)
    out_ref[...] = jnp.sum(h * w2t_ref[...], axis=1) + b2_ref[0]


def _mlp(x, w0, b0, w1, b1, w2t, b2):
    return pl.pallas_call(
        _mlp_body,
        grid=(BATCH_ // _BT,),
        in_specs=[
            pl.BlockSpec((_BT, 2 * EMB_), lambda i: (i, 0)),
            pl.BlockSpec((2 * EMB_, 256), lambda i: (0, 0)),
            pl.BlockSpec((256,), lambda i: (0,)),
            pl.BlockSpec((256, 64), lambda i: (0, 0)),
            pl.BlockSpec((64,), lambda i: (0,)),
            pl.BlockSpec((1, 64), lambda i: (0, 0)),
            pl.BlockSpec((1,), lambda i: (0,)),
        ],
        out_specs=pl.BlockSpec((_BT,), lambda i: (i,)),
        out_shape=jax.ShapeDtypeStruct((BATCH_,), jnp.float32),
    )(x, w0, b0, w1, b1, w2t, b2)


def kernel(user_id, movie_id, user_embeddings, movie_embeddings,
           W0, b0, W1, b1, W2, b2):
    x = _gather(user_id.astype(jnp.int32), movie_id.astype(jnp.int32),
                user_embeddings, movie_embeddings)
    return _mlp(x, W0, b0, W1, b1, W2.reshape(1, 64), b2)


# speedup vs baseline: 1.2792x; 1.2792x over previous
import jax
import jax.numpy as jnp
from jax import lax
from jax.experimental import pallas as pl
from jax.experimental.pallas import tpu as pltpu
from jax.experimental.pallas import tpu_sc as plsc

VOCAB_ = 10000
EMB_ = 32
BATCH_ = 16384

_NC = 2
_NS = 16
_NW = _NC * _NS
_BPW = BATCH_ // _NW
_CH = 128
_NCHUNK = _BPW // _CH


def _gather_body(uid_ref, mid_ref, utab_ref, mtab_ref, out_ref,
                 uidx, midx, urows, mrows, usem, msem):
    wid = lax.axis_index("s") * _NC + lax.axis_index("c")
    base = wid * _BPW
    pltpu.sync_copy(uid_ref.at[pl.ds(base, _BPW)], uidx)
    pltpu.sync_copy(mid_ref.at[pl.ds(base, _BPW)], midx)
    ucopies = [
        pltpu.async_copy(utab_ref.at[uidx.at[pl.ds(j * _CH, _CH)]],
                         urows.at[pl.ds(j * _CH, _CH)], usem)
        for j in range(_NCHUNK)
    ]
    mcopies = [
        pltpu.async_copy(mtab_ref.at[midx.at[pl.ds(j * _CH, _CH)]],
                         mrows.at[pl.ds(j * _CH, _CH)], msem)
        for j in range(_NCHUNK)
    ]
    for c in ucopies:
        c.wait()
    pltpu.sync_copy(urows, out_ref.at[pl.ds(base, _BPW), pl.ds(0, EMB_)])
    for c in mcopies:
        c.wait()
    pltpu.sync_copy(mrows, out_ref.at[pl.ds(base, _BPW), pl.ds(EMB_, EMB_)])


_gather = pl.kernel(
    _gather_body,
    mesh=plsc.VectorSubcoreMesh(core_axis_name="c", subcore_axis_name="s"),
    out_type=jax.ShapeDtypeStruct((BATCH_, 2 * EMB_), jnp.float32),
    scratch_types=[
        pltpu.VMEM((_BPW,), jnp.int32),
        pltpu.VMEM((_BPW,), jnp.int32),
        pltpu.VMEM((_BPW, EMB_), jnp.float32),
        pltpu.VMEM((_BPW, EMB_), jnp.float32),
        pltpu.SemaphoreType.DMA,
        pltpu.SemaphoreType.DMA,
    ],
    compiler_params=pltpu.CompilerParams(use_tc_tiling_on_sc=False),
)

_BT = 8192


def _mlp_body(x_ref, w0_ref, b0_ref, w1_ref, b1_ref, w2t_ref, b2_ref,
              out_ref):
    h = jnp.dot(x_ref[...], w0_ref[...], preferred_element_type=jnp.float32)
    h = jnp.maximum(h + b0_ref[...][None, :], 0.0)
    h = jnp.dot(h, w1_ref[...], preferred_element_type=jnp.float32)
    h = jnp.maximum(h + b1_ref[...][None, :], 0.0)
    out_ref[...] = jnp.sum(h * w2t_ref[...], axis=1, keepdims=True) + b2_ref[0]


def _mlp(x, w0, b0, w1, b1, w2t, b2):
    return pl.pallas_call(
        _mlp_body,
        grid=(BATCH_ // _BT,),
        in_specs=[
            pl.BlockSpec((_BT, 2 * EMB_), lambda i: (i, 0)),
            pl.BlockSpec((2 * EMB_, 256), lambda i: (0, 0)),
            pl.BlockSpec((256,), lambda i: (0,)),
            pl.BlockSpec((256, 64), lambda i: (0, 0)),
            pl.BlockSpec((64,), lambda i: (0,)),
            pl.BlockSpec((1, 64), lambda i: (0, 0)),
            pl.BlockSpec((1,), lambda i: (0,)),
        ],
        out_specs=pl.BlockSpec((_BT, 1), lambda i: (i, 0)),
        out_shape=jax.ShapeDtypeStruct((BATCH_, 1), jnp.float32),
    )(x, w0, b0, w1, b1, w2t, b2)


def kernel(user_id, movie_id, user_embeddings, movie_embeddings,
           W0, b0, W1, b1, W2, b2):
    x = _gather(user_id.astype(jnp.int32), movie_id.astype(jnp.int32),
                user_embeddings, movie_embeddings)
    return _mlp(x, W0, b0, W1, b1, W2.reshape(1, 64), b2).reshape(-1)


# speedup vs baseline: 1.3605x; 1.0635x over previous
import jax
import jax.numpy as jnp
from jax import lax
from jax.experimental import pallas as pl
from jax.experimental.pallas import tpu as pltpu
from jax.experimental.pallas import tpu_sc as plsc

VOCAB_ = 10000
EMB_ = 32
BATCH_ = 16384

_NC = 2
_NS = 16
_NW = _NC * _NS
_BPW = BATCH_ // _NW
_CH = 128
_NCHUNK = _BPW // _CH
_HALF = BATCH_ // 2


def _gather_body(uid_ref, mid_ref, utab_ref, mtab_ref, out_ref,
                 uidx, midx, urows, mrows, usem, msem):
    wid = lax.axis_index("s") * _NC + lax.axis_index("c")
    base = wid * _BPW
    row0 = base % _HALF
    col0 = (base // _HALF) * (2 * EMB_)
    pltpu.sync_copy(uid_ref.at[pl.ds(base, _BPW)], uidx)
    pltpu.sync_copy(mid_ref.at[pl.ds(base, _BPW)], midx)
    ucopies = [
        pltpu.async_copy(utab_ref.at[uidx.at[pl.ds(j * _CH, _CH)]],
                         urows.at[pl.ds(j * _CH, _CH)], usem)
        for j in range(_NCHUNK)
    ]
    mcopies = [
        pltpu.async_copy(mtab_ref.at[midx.at[pl.ds(j * _CH, _CH)]],
                         mrows.at[pl.ds(j * _CH, _CH)], msem)
        for j in range(_NCHUNK)
    ]
    for c in ucopies:
        c.wait()
    pltpu.sync_copy(urows, out_ref.at[pl.ds(row0, _BPW), pl.ds(col0, EMB_)])
    for c in mcopies:
        c.wait()
    pltpu.sync_copy(mrows,
                    out_ref.at[pl.ds(row0, _BPW), pl.ds(col0 + EMB_, EMB_)])


_gather = pl.kernel(
    _gather_body,
    mesh=plsc.VectorSubcoreMesh(core_axis_name="c", subcore_axis_name="s"),
    out_type=jax.ShapeDtypeStruct((_HALF, 128), jnp.float32),
    scratch_types=[
        pltpu.VMEM((_BPW,), jnp.int32),
        pltpu.VMEM((_BPW,), jnp.int32),
        pltpu.VMEM((_BPW, EMB_), jnp.float32),
        pltpu.VMEM((_BPW, EMB_), jnp.float32),
        pltpu.SemaphoreType.DMA,
        pltpu.SemaphoreType.DMA,
    ],
    compiler_params=pltpu.CompilerParams(use_tc_tiling_on_sc=False),
)

_BT = _HALF


def _mlp_body(x_ref, w0_ref, b0_ref, w1_ref, b1_ref, w2t_ref, b2_ref,
              out_ref):
    def head(xh):
        h = jnp.dot(xh, w0_ref[...], preferred_element_type=jnp.float32)
        h = jnp.maximum(h + b0_ref[...][None, :], 0.0)
        h = jnp.dot(h, w1_ref[...], preferred_element_type=jnp.float32)
        h = jnp.maximum(h + b1_ref[...][None, :], 0.0)
        return (jnp.sum(h * w2t_ref[...], axis=1, keepdims=True)
                + b2_ref[0])

    out_ref[0:_HALF, :] = head(x_ref[:, 0:2 * EMB_])
    out_ref[_HALF:BATCH_, :] = head(x_ref[:, 2 * EMB_:4 * EMB_])


def _mlp(x, w0, b0, w1, b1, w2t, b2):
    return pl.pallas_call(
        _mlp_body,
        in_specs=[
            pl.BlockSpec((_HALF, 128), lambda: (0, 0)),
            pl.BlockSpec((2 * EMB_, 256), lambda: (0, 0)),
            pl.BlockSpec((256,), lambda: (0,)),
            pl.BlockSpec((256, 64), lambda: (0, 0)),
            pl.BlockSpec((64,), lambda: (0,)),
            pl.BlockSpec((1, 64), lambda: (0, 0)),
            pl.BlockSpec((1,), lambda: (0,)),
        ],
        out_specs=pl.BlockSpec((BATCH_, 1), lambda: (0, 0)),
        out_shape=jax.ShapeDtypeStruct((BATCH_, 1), jnp.float32),
    )(x, w0, b0, w1, b1, w2t, b2)


def kernel(user_id, movie_id, user_embeddings, movie_embeddings,
           W0, b0, W1, b1, W2, b2):
    x = _gather(user_id.astype(jnp.int32), movie_id.astype(jnp.int32),
                user_embeddings, movie_embeddings)
    return _mlp(x, W0, b0, W1, b1, W2.reshape(1, 64), b2).reshape(-1)


# speedup vs baseline: 1.6988x; 1.2487x over previous
import jax
import jax.numpy as jnp
from jax import lax
from jax.experimental import pallas as pl
from jax.experimental.pallas import tpu as pltpu
from jax.experimental.pallas import tpu_sc as plsc

VOCAB_ = 10000
EMB_ = 32
BATCH_ = 16384

_NC = 2
_NS = 16
_NW = _NC * _NS
_BPW = BATCH_ // _NW
_CH = 128
_NCHUNK = _BPW // _CH
_HALF = BATCH_ // 2


def _gather_body(uid_ref, mid_ref, utab_ref, mtab_ref, out_ref,
                 uidx, midx, urows, mrows, usem, msem):
    wid = lax.axis_index("s") * _NC + lax.axis_index("c")
    base = wid * _BPW
    row0 = base % _HALF
    col0 = (base // _HALF) * (2 * EMB_)
    pltpu.sync_copy(uid_ref.at[pl.ds(base, _BPW)], uidx)
    pltpu.sync_copy(mid_ref.at[pl.ds(base, _BPW)], midx)
    ucopies = [
        pltpu.async_copy(utab_ref.at[uidx.at[pl.ds(j * _CH, _CH)]],
                         urows.at[pl.ds(j * _CH, _CH)], usem)
        for j in range(_NCHUNK)
    ]
    mcopies = [
        pltpu.async_copy(mtab_ref.at[midx.at[pl.ds(j * _CH, _CH)]],
                         mrows.at[pl.ds(j * _CH, _CH)], msem)
        for j in range(_NCHUNK)
    ]
    for c in ucopies:
        c.wait()
    pltpu.sync_copy(urows, out_ref.at[pl.ds(row0, _BPW), pl.ds(col0, EMB_)])
    for c in mcopies:
        c.wait()
    pltpu.sync_copy(mrows,
                    out_ref.at[pl.ds(row0, _BPW), pl.ds(col0 + EMB_, EMB_)])


_gather = pl.kernel(
    _gather_body,
    mesh=plsc.VectorSubcoreMesh(core_axis_name="c", subcore_axis_name="s"),
    out_type=jax.ShapeDtypeStruct((_HALF, 128), jnp.float32),
    scratch_types=[
        pltpu.VMEM((_BPW,), jnp.int32),
        pltpu.VMEM((_BPW,), jnp.int32),
        pltpu.VMEM((_BPW, EMB_), jnp.float32),
        pltpu.VMEM((_BPW, EMB_), jnp.float32),
        pltpu.SemaphoreType.DMA,
        pltpu.SemaphoreType.DMA,
    ],
    compiler_params=pltpu.CompilerParams(use_tc_tiling_on_sc=False),
)

_BT = _HALF


def _mlp_body(x_ref, w0_ref, b0_ref, w1_ref, b1_ref, w2t_ref, b2_ref,
              out_ref):
    def head(xh):
        h = jnp.dot(xh, w0_ref[...], preferred_element_type=jnp.float32)
        h = jnp.maximum(h + b0_ref[...][None, :], 0.0)
        h = jnp.dot(h, w1_ref[...], preferred_element_type=jnp.float32)
        h = jnp.maximum(h + b1_ref[...][None, :], 0.0)
        r = lax.dot_general(w2t_ref[...], h, (((1,), (1,)), ((), ())),
                            preferred_element_type=jnp.float32)
        return r[0] + b2_ref[0]

    out_ref[pl.ds(0, _HALF)] = head(x_ref[:, 0:2 * EMB_])
    out_ref[pl.ds(_HALF, _HALF)] = head(x_ref[:, 2 * EMB_:4 * EMB_])


def _mlp(x, w0, b0, w1, b1, w2t, b2):
    return pl.pallas_call(
        _mlp_body,
        in_specs=[
            pl.BlockSpec((_HALF, 128), lambda: (0, 0)),
            pl.BlockSpec((2 * EMB_, 256), lambda: (0, 0)),
            pl.BlockSpec((256,), lambda: (0,)),
            pl.BlockSpec((256, 64), lambda: (0, 0)),
            pl.BlockSpec((64,), lambda: (0,)),
            pl.BlockSpec((1, 64), lambda: (0, 0)),
            pl.BlockSpec((1,), lambda: (0,)),
        ],
        out_specs=pl.BlockSpec((BATCH_,), lambda: (0,)),
        out_shape=jax.ShapeDtypeStruct((BATCH_,), jnp.float32),
    )(x, w0, b0, w1, b1, w2t, b2)


def kernel(user_id, movie_id, user_embeddings, movie_embeddings,
           W0, b0, W1, b1, W2, b2):
    x = _gather(user_id.astype(jnp.int32), movie_id.astype(jnp.int32),
                user_embeddings, movie_embeddings)
    return _mlp(x, W0, b0, W1, b1, W2.reshape(1, 64), b2)
